# baseline (speedup 1.0000x reference)
import functools

import jax
import jax.numpy as jnp
from jax import lax
from jax.experimental import pallas as pl
from jax.experimental.pallas import tpu as pltpu
from jax.experimental.pallas import tpu_sc as plsc

B = 16384
D = 32
NC = 2
NS = 16
NW = NC * NS
BPW = B // NW

BLK = 2048


def _sc_gather(cust_ids, prod_ids, cust_emb, prod_emb, cust_bias, prod_bias,
               ce_out, pe_out, cb_out, pb_out,
               cidx_v, pidx_v, ce_v, pe_v, cb_v, pb_v, sem):
    wid = lax.axis_index("s") * NC + lax.axis_index("c")
    base = wid * BPW
    pltpu.sync_copy(cust_ids.at[pl.ds(base, BPW)], cidx_v)
    pltpu.sync_copy(prod_ids.at[pl.ds(base, BPW)], pidx_v)
    c1 = pltpu.async_copy(cust_emb.at[cidx_v], ce_v, sem)
    c2 = pltpu.async_copy(prod_emb.at[pidx_v], pe_v, sem)
    c3 = pltpu.async_copy(cust_bias.at[cidx_v], cb_v, sem)
    c4 = pltpu.async_copy(prod_bias.at[pidx_v], pb_v, sem)
    c1.wait()
    c2.wait()
    c3.wait()
    c4.wait()
    pltpu.sync_copy(ce_v, ce_out.at[pl.ds(base, BPW)])
    pltpu.sync_copy(pe_v, pe_out.at[pl.ds(base, BPW)])
    pltpu.sync_copy(cb_v, cb_out.at[pl.ds(base, BPW)])
    pltpu.sync_copy(pb_v, pb_out.at[pl.ds(base, BPW)])


def _dense_body(ce_ref, pe_ref, cb_ref, pb_ref, w1c_ref, w1p_ref, b1_ref,
                w2_ref, b2_ref, w3_ref, const_ref, out_ref):
    ce = ce_ref[...]
    pe = pe_ref[...]
    mf = jnp.sum(ce * pe, axis=1, keepdims=True)
    h1 = jnp.maximum(
        jnp.dot(ce, w1c_ref[...], preferred_element_type=jnp.float32)
        + jnp.dot(pe, w1p_ref[...], preferred_element_type=jnp.float32)
        + b1_ref[...], 0.0)
    h2 = jnp.maximum(
        jnp.dot(h1, w2_ref[...], preferred_element_type=jnp.float32)
        + b2_ref[...], 0.0)
    mlp = jnp.sum(h2 * w3_ref[...], axis=1, keepdims=True)
    logit = (0.6 * mf + 0.4 * mlp + cb_ref[...] + pb_ref[...]
             + const_ref[...])
    out_ref[...] = jax.nn.sigmoid(logit)


def kernel(customer_ids, product_ids, cust_emb, prod_emb, cust_bias,
           prod_bias, global_bias, W1, b1, W2, b2, W3, b3):
    cids = customer_ids.astype(jnp.int32)
    pids = product_ids.astype(jnp.int32)
    cbias = cust_bias.reshape(-1)
    pbias = prod_bias.reshape(-1)

    mesh = plsc.VectorSubcoreMesh(
        core_axis_name="c", subcore_axis_name="s",
        num_cores=NC, num_subcores=NS)
    sc_call = pl.kernel(
        _sc_gather,
        out_type=[
            jax.ShapeDtypeStruct((B, D), jnp.float32),
            jax.ShapeDtypeStruct((B, D), jnp.float32),
            jax.ShapeDtypeStruct((B,), jnp.float32),
            jax.ShapeDtypeStruct((B,), jnp.float32),
        ],
        mesh=mesh,
        scratch_types=[
            pltpu.VMEM((BPW,), jnp.int32),
            pltpu.VMEM((BPW,), jnp.int32),
            pltpu.VMEM((BPW, D), jnp.float32),
            pltpu.VMEM((BPW, D), jnp.float32),
            pltpu.VMEM((BPW,), jnp.float32),
            pltpu.VMEM((BPW,), jnp.float32),
            pltpu.SemaphoreType.DMA,
        ],
        compiler_params=pltpu.CompilerParams(use_tc_tiling_on_sc=False),
    )
    ce, pe, cb, pb = sc_call(cids, pids, cust_emb, prod_emb, cbias, pbias)

    w1c = W1[:D, :]
    w1p = W1[D:, :]
    const = (0.4 * b3 + global_bias).reshape(1, 1)

    grid = (B // BLK,)
    out = pl.pallas_call(
        _dense_body,
        grid=grid,
        in_specs=[
            pl.BlockSpec((BLK, D), lambda i: (i, 0)),
            pl.BlockSpec((BLK, D), lambda i: (i, 0)),
            pl.BlockSpec((BLK, 1), lambda i: (i, 0)),
            pl.BlockSpec((BLK, 1), lambda i: (i, 0)),
            pl.BlockSpec((D, 64), lambda i: (0, 0)),
            pl.BlockSpec((D, 64), lambda i: (0, 0)),
            pl.BlockSpec((1, 64), lambda i: (0, 0)),
            pl.BlockSpec((64, 32), lambda i: (0, 0)),
            pl.BlockSpec((1, 32), lambda i: (0, 0)),
            pl.BlockSpec((1, 32), lambda i: (0, 0)),
            pl.BlockSpec((1, 1), lambda i: (0, 0)),
        ],
        out_specs=pl.BlockSpec((BLK, 1), lambda i: (i, 0)),
        out_shape=jax.ShapeDtypeStruct((B, 1), jnp.float32),
    )(ce, pe, cb.reshape(B, 1), pb.reshape(B, 1), w1c, w1p,
      b1.reshape(1, 64), W2, b2.reshape(1, 32), W3.reshape(1, 32), const)
    return out.reshape(B)
